# baseline (speedup 1.0000x reference)
import functools

import jax
import jax.numpy as jnp
from jax import lax
from jax.experimental import pallas as pl
from jax.experimental.pallas import tpu as pltpu
from jax.experimental.pallas import tpu_sc as plsc

N_NODES = 50000
N_EDGES = 800000
H = 128
G = 512
NEG = 0.01

NW = 32
NPW = 1568
NPAD = NW * NPW
NSUB = 4
SUBN = NPW // NSUB
EPW = 25088
EPAD = NW * EPW
C = 128
OFFPAD = NPAD + 16


def _mesh():
    return plsc.VectorSubcoreMesh(core_axis_name="c", subcore_axis_name="s")


def _lrelu(v, s):
    return jnp.where(v >= 0, v, s * v)


def _elu(v):
    return jnp.where(v > 0, v, jnp.exp(jnp.minimum(v, 0.0)) - 1.0)


_BM = 512
_NBLK = NPAD // _BM
_BME = 1024
_NBLKE = EPAD // _BME


def _dot(a, b):
    return jnp.dot(a, b, preferred_element_type=jnp.float32)


def _row_spec(bm, ncols):
    return pl.BlockSpec((bm, ncols), lambda i: (i, 0))


def _full_spec(shape):
    return pl.BlockSpec(shape, lambda i: tuple(0 for _ in shape))


def _tc_lin1(x, w_t, b):
    def body(x_ref, w_ref, b_ref, o_ref):
        o_ref[...] = _lrelu(_dot(x_ref[...], w_ref[...]) + b_ref[...], 0.01)

    return pl.pallas_call(
        body,
        grid=(_NBLK,),
        in_specs=[_row_spec(_BM, 40), _full_spec((40, H)), _full_spec((1, H))],
        out_specs=_row_spec(_BM, H),
        out_shape=jax.ShapeDtypeStruct((NPAD, H), jnp.float32),
    )(x, w_t, b)


def _tc_pre_g(x0, w1x_t, g2_t, attr):
    def body(x_ref, a_ref, b_ref, r_ref, o1, o2, o3):
        x = x_ref[...]
        o1[...] = _dot(x, a_ref[...])
        o2[...] = _dot(x, b_ref[...])
        o3[...] = _dot(x, r_ref[...])

    return pl.pallas_call(
        body,
        grid=(_NBLK,),
        in_specs=[_row_spec(_BM, H), _full_spec((H, H)), _full_spec((H, H)),
                  _full_spec((H, 1))],
        out_specs=[_row_spec(_BM, H), _row_spec(_BM, H), _row_spec(_BM, 1)],
        out_shape=[jax.ShapeDtypeStruct((NPAD, H), jnp.float32),
                   jax.ShapeDtypeStruct((NPAD, H), jnp.float32),
                   jax.ShapeDtypeStruct((NPAD, 1), jnp.float32)],
    )(x0, w1x_t, g2_t, attr)


def _tc_edge_alpha(xw1g, ea, w1e_t, attl):
    def body(g_ref, e_ref, w_ref, a_ref, o_ref):
        u = g_ref[...] + _dot(e_ref[...], w_ref[...])
        o_ref[...] = _dot(_lrelu(u, 0.01), a_ref[...])

    return pl.pallas_call(
        body,
        grid=(_NBLKE,),
        in_specs=[_row_spec(_BME, H), _row_spec(_BME, 10),
                  _full_spec((10, H)), _full_spec((H, 1))],
        out_specs=_row_spec(_BME, 1),
        out_shape=jax.ShapeDtypeStruct((EPAD, 1), jnp.float32),
    )(xw1g, ea, w1e_t, attl)


def _tc_gru(hmsg, x, gbias, wih_t, whh_t, bih, bhh):
    def body(h_ref, x_ref, gb_ref, wi_ref, wh_ref, bi_ref, bh_ref, o_ref):
        h = _elu(h_ref[...] + gb_ref[...])
        x = x_ref[...]
        gi = _dot(h, wi_ref[...]) + bi_ref[...]
        gh = _dot(x, wh_ref[...]) + bh_ref[...]
        r = jax.nn.sigmoid(gi[:, 0:H] + gh[:, 0:H])
        z = jax.nn.sigmoid(gi[:, H:2 * H] + gh[:, H:2 * H])
        n = jnp.tanh(gi[:, 2 * H:3 * H] + r * gh[:, 2 * H:3 * H])
        o_ref[...] = jnp.maximum((1.0 - z) * n + z * x, 0.0)

    return pl.pallas_call(
        body,
        grid=(_NBLK,),
        in_specs=[_row_spec(_BM, H), _row_spec(_BM, H), _full_spec((1, H)),
                  _full_spec((H, 3 * H)), _full_spec((H, 3 * H)),
                  _full_spec((1, 3 * H)), _full_spec((1, 3 * H))],
        out_specs=_row_spec(_BM, H),
        out_shape=jax.ShapeDtypeStruct((NPAD, H), jnp.float32),
    )(hmsg, x, gbias, wih_t, whh_t, bih, bhh)


def _tc_gat_pre(x, w_t, att_src, att_dst):
    def body(x_ref, w_ref, s_ref, d_ref, o1, o2, o3):
        xt = _dot(x_ref[...], w_ref[...])
        o1[...] = xt
        o2[...] = _dot(xt, s_ref[...])
        o3[...] = _dot(xt, d_ref[...])

    return pl.pallas_call(
        body,
        grid=(_NBLK,),
        in_specs=[_row_spec(_BM, H), _full_spec((H, H)), _full_spec((H, 1)),
                  _full_spec((H, 1))],
        out_specs=[_row_spec(_BM, H), _row_spec(_BM, 1), _row_spec(_BM, 1)],
        out_shape=[jax.ShapeDtypeStruct((NPAD, H), jnp.float32),
                   jax.ShapeDtypeStruct((NPAD, 1), jnp.float32),
                   jax.ShapeDtypeStruct((NPAD, 1), jnp.float32)],
    )(x, w_t, att_src, att_dst)


def _tc_mol_pre(x, w_t, att_src):
    def body(x_ref, w_ref, s_ref, o1, o2):
        xs = _dot(x_ref[...], w_ref[...])
        o1[...] = xs
        o2[...] = _dot(xs, s_ref[...])

    return pl.pallas_call(
        body,
        grid=(_NBLK,),
        in_specs=[_row_spec(_BM, H), _full_spec((H, H)), _full_spec((H, 1))],
        out_specs=[_row_spec(_BM, H), _row_spec(_BM, 1)],
        out_shape=[jax.ShapeDtypeStruct((NPAD, H), jnp.float32),
                   jax.ShapeDtypeStruct((NPAD, 1), jnp.float32)],
    )(x, w_t, att_src)


def _tc_pool_init(pooled, vad):
    def body(p_ref, v_ref, o1, o2):
        o = jnp.maximum(p_ref[...], 0.0)
        o1[...] = o
        o2[...] = _dot(o, v_ref[...])

    return pl.pallas_call(
        body,
        grid=(1,),
        in_specs=[_row_spec(G, H), _full_spec((H, 1))],
        out_specs=[_row_spec(G, H), _row_spec(G, 1)],
        out_shape=[jax.ShapeDtypeStruct((G, H), jnp.float32),
                   jax.ShapeDtypeStruct((G, 1), jnp.float32)],
    )(pooled, vad)


def _tc_gru_mol(h, out, molbias, wih_t, whh_t, bih, bhh, vad):
    def body(h_ref, x_ref, mb_ref, wi_ref, wh_ref, bi_ref, bh_ref, v_ref,
             o1, o2):
        hh = _elu(h_ref[...] + mb_ref[...])
        x = x_ref[...]
        gi = _dot(hh, wi_ref[...]) + bi_ref[...]
        gh = _dot(x, wh_ref[...]) + bh_ref[...]
        r = jax.nn.sigmoid(gi[:, 0:H] + gh[:, 0:H])
        z = jax.nn.sigmoid(gi[:, H:2 * H] + gh[:, H:2 * H])
        n = jnp.tanh(gi[:, 2 * H:3 * H] + r * gh[:, 2 * H:3 * H])
        o = jnp.maximum((1.0 - z) * n + z * x, 0.0)
        o1[...] = o
        o2[...] = _dot(o, v_ref[...])

    return pl.pallas_call(
        body,
        grid=(1,),
        in_specs=[_row_spec(G, H), _row_spec(G, H), _full_spec((1, H)),
                  _full_spec((H, 3 * H)), _full_spec((H, 3 * H)),
                  _full_spec((1, 3 * H)), _full_spec((1, 3 * H)),
                  _full_spec((H, 1))],
        out_specs=[_row_spec(G, H), _row_spec(G, 1)],
        out_shape=[jax.ShapeDtypeStruct((G, H), jnp.float32),
                   jax.ShapeDtypeStruct((G, 1), jnp.float32)],
    )(h, out, molbias, wih_t, whh_t, bih, bhh, vad)


def _tc_head(out, l2w_t, l2b, h1w_t, h1b, h2w_t, h2b):
    def body(x_ref, aw, ab, bw, bb, cw, cb, o_ref):
        o = _dot(x_ref[...], aw[...]) + ab[...]
        y1 = jnp.maximum(_dot(o, bw[...]) + bb[...], 0.0)
        o_ref[...] = _dot(y1, cw[...]) + cb[...]

    return pl.pallas_call(
        body,
        grid=(1,),
        in_specs=[_row_spec(G, H), _full_spec((H, H)), _full_spec((1, H)),
                  _full_spec((H, 64)), _full_spec((1, 64)),
                  _full_spec((64, 1)), _full_spec((1, 1))],
        out_specs=_row_spec(G, 1),
        out_shape=jax.ShapeDtypeStruct((G, 1), jnp.float32),
    )(out, l2w_t, l2b, h1w_t, h1b, h2w_t, h2b)


def _sc_gather_rows(table, idx):
    @functools.partial(
        pl.kernel, mesh=_mesh(),
        out_type=jax.ShapeDtypeStruct((EPAD, H), jnp.float32),
        scratch_types=(
            [pltpu.VMEM((C,), jnp.int32)] * 2 +
            [pltpu.VMEM((C, H), jnp.float32)] * 2 +
            [pltpu.SemaphoreType.DMA] * 6
        ),
    )
    def k(tab, idx_hbm, out, idx0, idx1, rows0, rows1,
          is0, is1, gs0, gs1, ws0, ws1):
        idxs, rowss = [idx0, idx1], [rows0, rows1]
        isem, gsem, wsem = [is0, is1], [gs0, gs1], [ws0, ws1]
        wid = lax.axis_index("s") * 2 + lax.axis_index("c")
        base = pl.multiple_of(wid * EPW, 8)
        nch = EPW // C

        def e0_of(jc):
            return pl.multiple_of(base + jc * C, 8)

        def issue_idx(b, jc):
            pltpu.async_copy(idx_hbm.at[pl.ds(e0_of(jc), C)], idxs[b],
                             isem[b])

        def wait_idx(b):
            pltpu.make_async_copy(idx_hbm.at[pl.ds(e0_of(0), C)], idxs[b],
                                  isem[b]).wait()

        def issue_gath(b):
            pltpu.async_copy(tab.at[idxs[b]], rowss[b], gsem[b])

        def wait_gath(b):
            pltpu.make_async_copy(tab.at[idxs[b]], rowss[b], gsem[b]).wait()

        def issue_wb(b, jc):
            pltpu.async_copy(rowss[b], out.at[pl.ds(e0_of(jc), C)], wsem[b])

        def wait_wb(b):
            pltpu.make_async_copy(rowss[b], out.at[pl.ds(e0_of(0), C)],
                                  wsem[b]).wait()

        issue_idx(0, 0)
        wait_idx(0)
        issue_gath(0)
        issue_idx(1, 1)

        def outer(kk, _c):
            for b in range(2):
                j = 2 * kk + b
                q = 1 - b
                wait_idx(q)
                issue_gath(q)
                wait_gath(b)
                issue_wb(b, j)
                wait_wb(b)
                issue_idx(b, jnp.minimum(j + 2, nch - 1))
            return 0

        lax.fori_loop(0, nch // 2, outer, 0)
        wait_gath(0)
        issue_wb(0, nch - 1)
        wait_wb(0)
        wait_idx(1)

    return k(table, idx)


def _make_att_msg(g_mode):
    mesh = _mesh()

    @functools.partial(
        pl.kernel, mesh=mesh,
        out_type=jax.ShapeDtypeStruct((NPAD, H), jnp.float32),
        scratch_types=(
            [pltpu.VMEM((NPW + 32,), jnp.int32)] +
            [pltpu.VMEM((C,), jnp.int32)] * 6 +
            [pltpu.VMEM((C,), jnp.float32)] * 4 +
            [pltpu.VMEM((C + 16,), jnp.float32)] +
            [pltpu.VMEM((C + 16,), jnp.int32)] +
            [pltpu.VMEM((C, H), jnp.float32)] * 2 +
            [pltpu.VMEM((SUBN + 32,), jnp.float32)] +
            [pltpu.VMEM((SUBN + 1, H), jnp.float32)] +
            [pltpu.SemaphoreType.DMA] * 12
        ),
    )
    def k(tab, asrc, adst, srch, dsth, permh, offh, out,
          offv, src0, src1, dst0, dst1, perm0, perm1, as0, as1, ad0, ad1,
          wbuf, lnbuf, rows0, rows1, den, obuf,
          i00, i01, i02, i10, i11, i12, g00, g01, g02, g10, g11, g12):
        srcs, dsts, perms = [src0, src1], [dst0, dst1], [perm0, perm1]
        asvs, advs, rowss = [as0, as1], [ad0, ad1], [rows0, rows1]
        isem = [[i00, i01, i02], [i10, i11, i12]]
        gsem = [[g00, g01, g02], [g10, g11, g12]]
        wid = lax.axis_index("s") * 2 + lax.axis_index("c")
        n0 = pl.multiple_of(wid * NPW, 8)
        pltpu.sync_copy(offh.at[pl.ds(n0, NPW + 1)], offv.at[pl.ds(0, NPW + 1)])
        lane = lax.iota(jnp.int32, 16)

        def subchunk(s, _):
            d0 = pl.multiple_of(n0 + s * SUBN, 8)

            def zb(i, _c):
                for c in range(H // 16):
                    obuf[i, pl.ds(c * 16, 16)] = jnp.zeros((16,), jnp.float32)
                return 0
            lax.fori_loop(0, SUBN + 1, zb, 0)

            def zd(i, _c):
                den[pl.ds(i * 16, 16)] = jnp.zeros((16,), jnp.float32)
                return 0
            lax.fori_loop(0, (SUBN + 32) // 16, zd, 0)

            ls = s * SUBN
            e_start = offv[pl.ds(ls, 16)][0]
            e_end = offv[pl.ds(ls + SUBN, 16)][0]
            c0 = e_start - lax.rem(e_start, 8)
            nch = lax.div(e_end - c0 + (C - 1), C)
            nchm1 = jnp.maximum(nch - 1, 0)

            def e0_of(jc):
                return pl.multiple_of(c0 + jc * C, 8)

            def issue_idx(b, jc):
                e0 = e0_of(jc)
                pltpu.async_copy(srch.at[pl.ds(e0, C)], srcs[b], isem[b][0])
                pltpu.async_copy(dsth.at[pl.ds(e0, C)], dsts[b], isem[b][1])
                if g_mode:
                    pltpu.async_copy(asrc.at[pl.ds(e0, C)], asvs[b],
                                     isem[b][2])

            def wait_idx(b):
                e0 = e0_of(0)
                pltpu.make_async_copy(srch.at[pl.ds(e0, C)], srcs[b],
                                      isem[b][0]).wait()
                pltpu.make_async_copy(dsth.at[pl.ds(e0, C)], dsts[b],
                                      isem[b][1]).wait()
                if g_mode:
                    pltpu.make_async_copy(asrc.at[pl.ds(e0, C)], asvs[b],
                                          isem[b][2]).wait()

            def issue_gath(b):
                if not g_mode:
                    pltpu.async_copy(asrc.at[srcs[b]], asvs[b], gsem[b][0])
                pltpu.async_copy(adst.at[dsts[b]], advs[b], gsem[b][1])
                pltpu.async_copy(tab.at[srcs[b]], rowss[b], gsem[b][2])

            def wait_gath(b):
                if not g_mode:
                    pltpu.make_async_copy(asrc.at[srcs[b]], asvs[b],
                                          gsem[b][0]).wait()
                pltpu.make_async_copy(adst.at[dsts[b]], advs[b],
                                      gsem[b][1]).wait()
                pltpu.make_async_copy(tab.at[srcs[b]], rowss[b],
                                      gsem[b][2]).wait()

            def compute(b, j):
                jc = jnp.minimum(j, nchm1)
                e0 = e0_of(jc)
                eff_end = jnp.where(j < nch, e_end, e_start)
                asv, adv, dstv, rows = asvs[b], advs[b], dsts[b], rowss[b]
                for g in range(C // 16):
                    sl = pl.ds(g * 16, 16)
                    eidx = e0 + g * 16 + lane
                    al = asv[sl] + adv[sl]
                    al = jnp.where(al >= 0, al, NEG * al)
                    ex = jnp.exp(al)
                    valid = (eidx >= e_start) & (eidx < eff_end)
                    wbuf[sl] = jnp.where(valid, ex, 0.0)
                    ln = dstv[sl] - d0
                    lnbuf[sl] = jnp.where(valid, ln, jnp.int32(SUBN))

                def acc(e, _a):
                    w_e = wbuf[pl.ds(e, 16)][0]
                    ln_e = lnbuf[pl.ds(e, 16)][0]
                    dv = den[pl.ds(ln_e, 16)]
                    den[pl.ds(ln_e, 16)] = jnp.where(lane == 0, dv + w_e, dv)
                    for c in range(H // 16):
                        sl = pl.ds(c * 16, 16)
                        obuf[ln_e, sl] = obuf[ln_e, sl] + w_e * rows[e, sl]
                    return 0

                lax.fori_loop(0, C, acc, 0, unroll=4)

            @pl.when(nch > 0)
            def _pipeline():
                issue_idx(0, 0)
                wait_idx(0)
                issue_gath(0)
                issue_idx(1, jnp.minimum(1, nchm1))

                def outer(kk, _c):
                    for b in range(2):
                        j = 2 * kk + b
                        q = 1 - b
                        wait_idx(q)
                        issue_gath(q)
                        wait_gath(b)
                        issue_idx(b, jnp.minimum(j + 2, nchm1))
                        compute(b, j)
                    return 0

                lax.fori_loop(0, lax.div(nch + 1, 2), outer, 0)
                wait_gath(0)
                wait_idx(1)

            def recip(gq, _c):
                sl = pl.ds(gq * 16, 16)
                den[sl] = 1.0 / (den[sl] + 1e-16)
                return 0
            lax.fori_loop(0, (SUBN + 16) // 16, recip, 0)

            def scale(i, _c):
                inv = den[pl.ds(i, 16)][0]
                for c in range(H // 16):
                    sl = pl.ds(c * 16, 16)
                    obuf[i, sl] = obuf[i, sl] * inv
                return 0
            lax.fori_loop(0, SUBN, scale, 0)

            pltpu.sync_copy(obuf.at[pl.ds(0, SUBN)], out.at[pl.ds(d0, SUBN)])
            return 0

        lax.fori_loop(0, NSUB, subchunk, 0)

    return k


_att_msg_gat = _make_att_msg(False)
_att_msg_g = _make_att_msg(True)


def _make_seg_pool(att_mode):
    mesh = _mesh()
    GPW = G // NW

    @functools.partial(
        pl.kernel, mesh=mesh,
        out_type=jax.ShapeDtypeStruct((G, H), jnp.float32),
        scratch_types=[
            pltpu.VMEM((GPW + 17,), jnp.int32),
            pltpu.VMEM((32,), jnp.float32),
            pltpu.VMEM((C,), jnp.float32),
            pltpu.VMEM((C + 16,), jnp.float32),
            pltpu.VMEM((C, H), jnp.float32),
            pltpu.VMEM((32,), jnp.float32),
            pltpu.VMEM((GPW, H), jnp.float32),
            pltpu.SemaphoreType.DMA,
        ],
    )
    def k(xs, asn, adg, offh, out, offv, adgv, asv, wbuf, rows, den, obuf,
          sem):
        wid = lax.axis_index("s") * 2 + lax.axis_index("c")
        g0 = pl.multiple_of(wid * GPW, 8)
        pltpu.sync_copy(offh.at[pl.ds(g0, GPW + 1)], offv.at[pl.ds(0, GPW + 1)])
        if att_mode:
            pltpu.sync_copy(adg.at[pl.ds(g0, GPW)], adgv.at[pl.ds(0, GPW)])
        lane = lax.iota(jnp.int32, 16)

        def zb(i, _c):
            for c in range(H // 16):
                obuf[i, pl.ds(c * 16, 16)] = jnp.zeros((16,), jnp.float32)
            return 0
        lax.fori_loop(0, GPW, zb, 0)

        def graph(gi, _c):
            b0 = offv[pl.ds(gi, 16)][0]
            b1 = offv[pl.ds(gi + 1, 16)][0]
            if att_mode:
                ad_g = adgv[pl.ds(gi, 16)][0]
            else:
                ad_g = 0.0
            c0 = b0 - lax.rem(b0, 8)
            nch = lax.div(b1 - c0 + (C - 1), C)

            def chunk(j, dsum):
                e0 = pl.multiple_of(c0 + j * C, 8)
                pltpu.sync_copy(xs.at[pl.ds(e0, C)], rows)
                if att_mode:
                    pltpu.sync_copy(asn.at[pl.ds(e0, C)], asv)
                for g in range(C // 16):
                    sl = pl.ds(g * 16, 16)
                    nidx = e0 + g * 16 + lane
                    valid = (nidx >= b0) & (nidx < b1)
                    if att_mode:
                        al = asv[sl] + ad_g
                        al = jnp.where(al >= 0, al, NEG * al)
                        ex = jnp.exp(al)
                    else:
                        ex = jnp.ones((16,), jnp.float32)
                    wbuf[sl] = jnp.where(valid, ex, 0.0)

                def acc(e, da):
                    w_e = wbuf[pl.ds(e, 16)][0]
                    for c in range(H // 16):
                        sl = pl.ds(c * 16, 16)
                        obuf[gi, sl] = obuf[gi, sl] + w_e * rows[e, sl]
                    return da + w_e

                return lax.fori_loop(0, C, acc, dsum, unroll=4)

            dsum = lax.fori_loop(0, nch, chunk, 0.0)
            dv = den[pl.ds(gi, 16)]
            den[pl.ds(gi, 16)] = jnp.where(lane == 0, dsum, dv)
            return 0

        lax.fori_loop(0, GPW, graph, 0)

        if att_mode:
            dv = den[pl.ds(0, 16)]
            den[pl.ds(0, 16)] = 1.0 / (dv + 1e-16)

            def scale(i, _c):
                inv = den[pl.ds(i, 16)][0]
                for c in range(H // 16):
                    sl = pl.ds(c * 16, 16)
                    obuf[i, sl] = obuf[i, sl] * inv
                return 0
            lax.fori_loop(0, GPW, scale, 0)

        pltpu.sync_copy(obuf, out.at[pl.ds(g0, GPW)])

    return k


_seg_pool_sum = _make_seg_pool(False)
_seg_pool_att = _make_seg_pool(True)


def kernel(x, edge_index, edge_attr, batch, params):
    p = params
    src = edge_index[0]
    dst = edge_index[1]

    eiota = lax.iota(jnp.int32, N_EDGES)
    dst_s, perm = lax.sort([dst, eiota], num_keys=1)
    src_s = jnp.take(src, perm)
    counts = jax.ops.segment_sum(jnp.ones((N_EDGES,), jnp.int32), dst,
                                 num_segments=N_NODES)
    offsets = jnp.concatenate([jnp.zeros((1,), jnp.int32),
                               jnp.cumsum(counts, dtype=jnp.int32)])
    offsets = jnp.pad(offsets, (0, OFFPAD - (N_NODES + 1)),
                      constant_values=N_EDGES)
    bcounts = jax.ops.segment_sum(jnp.ones((N_NODES,), jnp.int32), batch,
                                  num_segments=G)
    boffsets = jnp.concatenate([jnp.zeros((1,), jnp.int32),
                                jnp.cumsum(bcounts, dtype=jnp.int32)])
    boffsets = jnp.pad(boffsets, (0, 32), constant_values=N_NODES)

    epad = EPAD - N_EDGES
    src_s = jnp.pad(src_s, (0, epad))
    dst_s = jnp.pad(dst_s, (0, epad), constant_values=N_NODES - 8)
    src_o = jnp.pad(src, (0, epad))
    ea = jnp.pad(edge_attr, ((0, epad), (0, 0)))

    xpad = jnp.pad(x, ((0, NPAD - N_NODES), (0, 1)))

    lin1_wt = jnp.pad(p['lin1_W'].T, ((0, 1), (0, 0)))
    w1x_t = p['g_lin1_W'][:, :H].T
    w1e_t = p['g_lin1_W'][:, H:].T
    g2_t = p['g_lin2_W'].T
    vad = p['mol_lin_W'].T @ p['mol_att_dst']

    def r2(v):
        return v.reshape(1, -1)

    def c2(v):
        return v.reshape(-1, 1)

    x0 = _tc_lin1(xpad, lin1_wt, r2(p['lin1_b']))
    xw1, xg2, ar1 = _tc_pre_g(x0, w1x_t, g2_t, c2(p['g_att_r']))

    xw1g = _sc_gather_rows(xw1, src_o)
    b_edge = _tc_edge_alpha(xw1g, ea, w1e_t, c2(p['g_att_l']))

    b_sorted = jnp.take(b_edge.reshape(-1), perm)
    b_sorted = jnp.pad(b_sorted, (0, epad))
    msg = _att_msg_g(xg2, b_sorted, ar1.reshape(-1),
                     src_s, dst_s, src_s, offsets)
    xcur = _tc_gru(msg, x0, r2(p['g_bias']),
                   p['gru0_Wih'].T, p['gru0_Whh'].T,
                   r2(p['gru0_bih']), r2(p['gru0_bhh']))

    for l in range(2):
        xt, a_s, a_d = _tc_gat_pre(xcur, p['a%d_lin_W' % l].T,
                                   c2(p['a%d_att_src' % l]),
                                   c2(p['a%d_att_dst' % l]))
        msg = _att_msg_gat(xt, a_s.reshape(-1), a_d.reshape(-1),
                           src_s, dst_s, src_s, offsets)
        gname = 'gru%d' % (l + 1)
        xcur = _tc_gru(msg, xcur, r2(p['a%d_bias' % l]),
                       p[gname + '_Wih'].T, p[gname + '_Whh'].T,
                       r2(p[gname + '_bih']), r2(p[gname + '_bhh']))

    zeros_n = jnp.zeros((NPAD,), jnp.float32)
    pooled = _seg_pool_sum(xcur, zeros_n, jnp.zeros((G,), jnp.float32),
                           boffsets)
    out, ad = _tc_pool_init(pooled, c2(vad))
    xs, asn = _tc_mol_pre(xcur, p['mol_lin_W'].T, c2(p['mol_att_src']))
    for _t in range(2):
        h = _seg_pool_att(xs, asn.reshape(-1), ad.reshape(-1), boffsets)
        out, ad = _tc_gru_mol(h, out, r2(p['mol_bias']),
                              p['gru3_Wih'].T, p['gru3_Whh'].T,
                              r2(p['gru3_bih']), r2(p['gru3_bhh']),
                              c2(vad))

    return _tc_head(out, p['lin2_W'].T, r2(p['lin2_b']),
                    p['head_W1'].T, r2(p['head_b1']),
                    p['head_W2'].T, r2(p['head_b2']))


# speedup vs baseline: 1.0542x; 1.0542x over previous
import functools

import jax
import jax.numpy as jnp
from jax import lax
from jax.experimental import pallas as pl
from jax.experimental.pallas import tpu as pltpu
from jax.experimental.pallas import tpu_sc as plsc

N_NODES = 50000
N_EDGES = 800000
H = 128
G = 512
NEG = 0.01

NW = 32
NPW = 1568
NPAD = NW * NPW
NSUB = 4
SUBN = NPW // NSUB
EPW = 25088
EPAD = NW * EPW
C = 128
OFFPAD = NPAD + 16


def _mesh():
    return plsc.VectorSubcoreMesh(core_axis_name="c", subcore_axis_name="s")


def _lrelu(v, s):
    return jnp.where(v >= 0, v, s * v)


def _elu(v):
    return jnp.where(v > 0, v, jnp.exp(jnp.minimum(v, 0.0)) - 1.0)


_BM = 512
_NBLK = NPAD // _BM
_BME = 1024
_NBLKE = EPAD // _BME


def _dot(a, b):
    return jnp.dot(a, b, preferred_element_type=jnp.float32)


def _row_spec(bm, ncols):
    return pl.BlockSpec((bm, ncols), lambda i: (i, 0))


def _full_spec(shape):
    return pl.BlockSpec(shape, lambda i: tuple(0 for _ in shape))


def _tc_lin1(x, w_t, b):
    def body(x_ref, w_ref, b_ref, o_ref):
        o_ref[...] = _lrelu(_dot(x_ref[...], w_ref[...]) + b_ref[...], 0.01)

    return pl.pallas_call(
        body,
        grid=(_NBLK,),
        in_specs=[_row_spec(_BM, 40), _full_spec((40, H)), _full_spec((1, H))],
        out_specs=_row_spec(_BM, H),
        out_shape=jax.ShapeDtypeStruct((NPAD, H), jnp.float32),
    )(x, w_t, b)


def _tc_pre_g(x0, w1x_t, g2_t, attr):
    def body(x_ref, a_ref, b_ref, r_ref, o1, o2, o3):
        x = x_ref[...]
        o1[...] = _dot(x, a_ref[...])
        o2[...] = _dot(x, b_ref[...])
        o3[...] = _dot(x, r_ref[...])

    return pl.pallas_call(
        body,
        grid=(_NBLK,),
        in_specs=[_row_spec(_BM, H), _full_spec((H, H)), _full_spec((H, H)),
                  _full_spec((H, 1))],
        out_specs=[_row_spec(_BM, H), _row_spec(_BM, H), _row_spec(_BM, 1)],
        out_shape=[jax.ShapeDtypeStruct((NPAD, H), jnp.float32),
                   jax.ShapeDtypeStruct((NPAD, H), jnp.float32),
                   jax.ShapeDtypeStruct((NPAD, 1), jnp.float32)],
    )(x0, w1x_t, g2_t, attr)


def _tc_edge_alpha(xw1g, ea, w1e_t, attl):
    def body(g_ref, e_ref, w_ref, a_ref, o_ref):
        u = g_ref[...] + _dot(e_ref[...], w_ref[...])
        o_ref[...] = _dot(_lrelu(u, 0.01), a_ref[...])

    return pl.pallas_call(
        body,
        grid=(_NBLKE,),
        in_specs=[_row_spec(_BME, H), _row_spec(_BME, 10),
                  _full_spec((10, H)), _full_spec((H, 1))],
        out_specs=_row_spec(_BME, 1),
        out_shape=jax.ShapeDtypeStruct((EPAD, 1), jnp.float32),
    )(xw1g, ea, w1e_t, attl)


def _tc_gru(hmsg, x, gbias, wih_t, whh_t, bih, bhh):
    def body(h_ref, x_ref, gb_ref, wi_ref, wh_ref, bi_ref, bh_ref, o_ref):
        h = _elu(h_ref[...] + gb_ref[...])
        x = x_ref[...]
        gi = _dot(h, wi_ref[...]) + bi_ref[...]
        gh = _dot(x, wh_ref[...]) + bh_ref[...]
        r = jax.nn.sigmoid(gi[:, 0:H] + gh[:, 0:H])
        z = jax.nn.sigmoid(gi[:, H:2 * H] + gh[:, H:2 * H])
        n = jnp.tanh(gi[:, 2 * H:3 * H] + r * gh[:, 2 * H:3 * H])
        o_ref[...] = jnp.maximum((1.0 - z) * n + z * x, 0.0)

    return pl.pallas_call(
        body,
        grid=(_NBLK,),
        in_specs=[_row_spec(_BM, H), _row_spec(_BM, H), _full_spec((1, H)),
                  _full_spec((H, 3 * H)), _full_spec((H, 3 * H)),
                  _full_spec((1, 3 * H)), _full_spec((1, 3 * H))],
        out_specs=_row_spec(_BM, H),
        out_shape=jax.ShapeDtypeStruct((NPAD, H), jnp.float32),
    )(hmsg, x, gbias, wih_t, whh_t, bih, bhh)


def _tc_gat_pre(x, w_t, att_src, att_dst):
    def body(x_ref, w_ref, s_ref, d_ref, o1, o2, o3):
        xt = _dot(x_ref[...], w_ref[...])
        o1[...] = xt
        o2[...] = _dot(xt, s_ref[...])
        o3[...] = _dot(xt, d_ref[...])

    return pl.pallas_call(
        body,
        grid=(_NBLK,),
        in_specs=[_row_spec(_BM, H), _full_spec((H, H)), _full_spec((H, 1)),
                  _full_spec((H, 1))],
        out_specs=[_row_spec(_BM, H), _row_spec(_BM, 1), _row_spec(_BM, 1)],
        out_shape=[jax.ShapeDtypeStruct((NPAD, H), jnp.float32),
                   jax.ShapeDtypeStruct((NPAD, 1), jnp.float32),
                   jax.ShapeDtypeStruct((NPAD, 1), jnp.float32)],
    )(x, w_t, att_src, att_dst)


def _tc_mol_pre(x, w_t, att_src):
    def body(x_ref, w_ref, s_ref, o1, o2):
        xs = _dot(x_ref[...], w_ref[...])
        o1[...] = xs
        o2[...] = _dot(xs, s_ref[...])

    return pl.pallas_call(
        body,
        grid=(_NBLK,),
        in_specs=[_row_spec(_BM, H), _full_spec((H, H)), _full_spec((H, 1))],
        out_specs=[_row_spec(_BM, H), _row_spec(_BM, 1)],
        out_shape=[jax.ShapeDtypeStruct((NPAD, H), jnp.float32),
                   jax.ShapeDtypeStruct((NPAD, 1), jnp.float32)],
    )(x, w_t, att_src)


def _tc_pool_init(pooled, vad):
    def body(p_ref, v_ref, o1, o2):
        o = jnp.maximum(p_ref[...], 0.0)
        o1[...] = o
        o2[...] = _dot(o, v_ref[...])

    return pl.pallas_call(
        body,
        grid=(1,),
        in_specs=[_row_spec(G, H), _full_spec((H, 1))],
        out_specs=[_row_spec(G, H), _row_spec(G, 1)],
        out_shape=[jax.ShapeDtypeStruct((G, H), jnp.float32),
                   jax.ShapeDtypeStruct((G, 1), jnp.float32)],
    )(pooled, vad)


def _tc_gru_mol(h, out, molbias, wih_t, whh_t, bih, bhh, vad):
    def body(h_ref, x_ref, mb_ref, wi_ref, wh_ref, bi_ref, bh_ref, v_ref,
             o1, o2):
        hh = _elu(h_ref[...] + mb_ref[...])
        x = x_ref[...]
        gi = _dot(hh, wi_ref[...]) + bi_ref[...]
        gh = _dot(x, wh_ref[...]) + bh_ref[...]
        r = jax.nn.sigmoid(gi[:, 0:H] + gh[:, 0:H])
        z = jax.nn.sigmoid(gi[:, H:2 * H] + gh[:, H:2 * H])
        n = jnp.tanh(gi[:, 2 * H:3 * H] + r * gh[:, 2 * H:3 * H])
        o = jnp.maximum((1.0 - z) * n + z * x, 0.0)
        o1[...] = o
        o2[...] = _dot(o, v_ref[...])

    return pl.pallas_call(
        body,
        grid=(1,),
        in_specs=[_row_spec(G, H), _row_spec(G, H), _full_spec((1, H)),
                  _full_spec((H, 3 * H)), _full_spec((H, 3 * H)),
                  _full_spec((1, 3 * H)), _full_spec((1, 3 * H)),
                  _full_spec((H, 1))],
        out_specs=[_row_spec(G, H), _row_spec(G, 1)],
        out_shape=[jax.ShapeDtypeStruct((G, H), jnp.float32),
                   jax.ShapeDtypeStruct((G, 1), jnp.float32)],
    )(h, out, molbias, wih_t, whh_t, bih, bhh, vad)


def _tc_head(out, l2w_t, l2b, h1w_t, h1b, h2w_t, h2b):
    def body(x_ref, aw, ab, bw, bb, cw, cb, o_ref):
        o = _dot(x_ref[...], aw[...]) + ab[...]
        y1 = jnp.maximum(_dot(o, bw[...]) + bb[...], 0.0)
        o_ref[...] = _dot(y1, cw[...]) + cb[...]

    return pl.pallas_call(
        body,
        grid=(1,),
        in_specs=[_row_spec(G, H), _full_spec((H, H)), _full_spec((1, H)),
                  _full_spec((H, 64)), _full_spec((1, 64)),
                  _full_spec((64, 1)), _full_spec((1, 1))],
        out_specs=_row_spec(G, 1),
        out_shape=jax.ShapeDtypeStruct((G, 1), jnp.float32),
    )(out, l2w_t, l2b, h1w_t, h1b, h2w_t, h2b)


def _sc_gather_rows(table, idx):
    @functools.partial(
        pl.kernel, mesh=_mesh(),
        out_type=jax.ShapeDtypeStruct((EPAD, H), jnp.float32),
        scratch_types=(
            [pltpu.VMEM((C,), jnp.int32)] * 2 +
            [pltpu.VMEM((C, H), jnp.float32)] * 2 +
            [pltpu.SemaphoreType.DMA] * 6
        ),
    )
    def k(tab, idx_hbm, out, idx0, idx1, rows0, rows1,
          is0, is1, gs0, gs1, ws0, ws1):
        idxs, rowss = [idx0, idx1], [rows0, rows1]
        isem, gsem, wsem = [is0, is1], [gs0, gs1], [ws0, ws1]
        wid = lax.axis_index("s") * 2 + lax.axis_index("c")
        base = pl.multiple_of(wid * EPW, 8)
        nch = EPW // C

        def e0_of(jc):
            return pl.multiple_of(base + jc * C, 8)

        def issue_idx(b, jc):
            pltpu.async_copy(idx_hbm.at[pl.ds(e0_of(jc), C)], idxs[b],
                             isem[b])

        def wait_idx(b):
            pltpu.make_async_copy(idx_hbm.at[pl.ds(e0_of(0), C)], idxs[b],
                                  isem[b]).wait()

        def issue_gath(b):
            pltpu.async_copy(tab.at[idxs[b]], rowss[b], gsem[b])

        def wait_gath(b):
            pltpu.make_async_copy(tab.at[idxs[b]], rowss[b], gsem[b]).wait()

        def issue_wb(b, jc):
            pltpu.async_copy(rowss[b], out.at[pl.ds(e0_of(jc), C)], wsem[b])

        def wait_wb(b):
            pltpu.make_async_copy(rowss[b], out.at[pl.ds(e0_of(0), C)],
                                  wsem[b]).wait()

        issue_idx(0, 0)
        wait_idx(0)
        issue_gath(0)
        issue_idx(1, 1)

        def outer(kk, _c):
            for b in range(2):
                j = 2 * kk + b
                q = 1 - b
                wait_idx(q)
                issue_gath(q)
                wait_gath(b)
                issue_wb(b, j)
                wait_wb(b)
                issue_idx(b, jnp.minimum(j + 2, nch - 1))
            return 0

        lax.fori_loop(0, nch // 2, outer, 0)
        wait_gath(0)
        issue_wb(0, nch - 1)
        wait_wb(0)
        wait_idx(1)

    return k(table, idx)


def _make_att_msg(g_mode):
    mesh = _mesh()

    @functools.partial(
        pl.kernel, mesh=mesh,
        out_type=jax.ShapeDtypeStruct((NPAD, H), jnp.float32),
        scratch_types=(
            [pltpu.VMEM((NPW + 32,), jnp.int32)] +
            [pltpu.VMEM((C,), jnp.int32)] * 6 +
            [pltpu.VMEM((C,), jnp.float32)] * 4 +
            [pltpu.VMEM((C + 16,), jnp.float32)] +
            [pltpu.VMEM((C + 16,), jnp.int32)] +
            [pltpu.VMEM((C, H), jnp.float32)] * 2 +
            [pltpu.VMEM((SUBN + 32,), jnp.float32)] +
            [pltpu.VMEM((SUBN + 1, H), jnp.float32)] +
            [pltpu.SemaphoreType.DMA] * 12
        ),
    )
    def k(tab, asrc, adst, srch, dsth, permh, offh, out,
          offv, src0, src1, dst0, dst1, perm0, perm1, as0, as1, ad0, ad1,
          wbuf, lnbuf, rows0, rows1, den, obuf,
          i00, i01, i02, i10, i11, i12, g00, g01, g02, g10, g11, g12):
        srcs, dsts, perms = [src0, src1], [dst0, dst1], [perm0, perm1]
        asvs, advs, rowss = [as0, as1], [ad0, ad1], [rows0, rows1]
        isem = [[i00, i01, i02], [i10, i11, i12]]
        gsem = [[g00, g01, g02], [g10, g11, g12]]
        wid = lax.axis_index("s") * 2 + lax.axis_index("c")
        n0 = pl.multiple_of(wid * NPW, 8)
        pltpu.sync_copy(offh.at[pl.ds(n0, NPW + 1)], offv.at[pl.ds(0, NPW + 1)])
        lane = lax.iota(jnp.int32, 16)

        def subchunk(s, _):
            d0 = pl.multiple_of(n0 + s * SUBN, 8)

            def zb(i, _c):
                for c in range(H // 16):
                    obuf[i, pl.ds(c * 16, 16)] = jnp.zeros((16,), jnp.float32)
                return 0
            lax.fori_loop(0, SUBN + 1, zb, 0)

            def zd(i, _c):
                den[pl.ds(i * 16, 16)] = jnp.zeros((16,), jnp.float32)
                return 0
            lax.fori_loop(0, (SUBN + 32) // 16, zd, 0)

            ls = s * SUBN
            e_start = offv[pl.ds(ls, 16)][0]
            e_end = offv[pl.ds(ls + SUBN, 16)][0]
            c0 = e_start - lax.rem(e_start, 8)
            nch = lax.div(e_end - c0 + (C - 1), C)
            nchm1 = jnp.maximum(nch - 1, 0)

            def e0_of(jc):
                return pl.multiple_of(c0 + jc * C, 8)

            def issue_idx(b, jc):
                e0 = e0_of(jc)
                pltpu.async_copy(srch.at[pl.ds(e0, C)], srcs[b], isem[b][0])
                pltpu.async_copy(dsth.at[pl.ds(e0, C)], dsts[b], isem[b][1])
                if g_mode:
                    pltpu.async_copy(permh.at[pl.ds(e0, C)], perms[b],
                                     isem[b][2])

            def wait_idx(b):
                e0 = e0_of(0)
                pltpu.make_async_copy(srch.at[pl.ds(e0, C)], srcs[b],
                                      isem[b][0]).wait()
                pltpu.make_async_copy(dsth.at[pl.ds(e0, C)], dsts[b],
                                      isem[b][1]).wait()
                if g_mode:
                    pltpu.make_async_copy(permh.at[pl.ds(e0, C)], perms[b],
                                          isem[b][2]).wait()

            def issue_gath(b):
                if g_mode:
                    pltpu.async_copy(asrc.at[perms[b]], asvs[b], gsem[b][0])
                else:
                    pltpu.async_copy(asrc.at[srcs[b]], asvs[b], gsem[b][0])
                pltpu.async_copy(adst.at[dsts[b]], advs[b], gsem[b][1])
                pltpu.async_copy(tab.at[srcs[b]], rowss[b], gsem[b][2])

            def wait_gath(b):
                if g_mode:
                    pltpu.make_async_copy(asrc.at[perms[b]], asvs[b],
                                          gsem[b][0]).wait()
                else:
                    pltpu.make_async_copy(asrc.at[srcs[b]], asvs[b],
                                          gsem[b][0]).wait()
                pltpu.make_async_copy(adst.at[dsts[b]], advs[b],
                                      gsem[b][1]).wait()
                pltpu.make_async_copy(tab.at[srcs[b]], rowss[b],
                                      gsem[b][2]).wait()

            def compute(b, j):
                jc = jnp.minimum(j, nchm1)
                e0 = e0_of(jc)
                eff_end = jnp.where(j < nch, e_end, e_start)
                asv, adv, dstv, rows = asvs[b], advs[b], dsts[b], rowss[b]
                for g in range(C // 16):
                    sl = pl.ds(g * 16, 16)
                    eidx = e0 + g * 16 + lane
                    al = asv[sl] + adv[sl]
                    al = jnp.where(al >= 0, al, NEG * al)
                    ex = jnp.exp(al)
                    valid = (eidx >= e_start) & (eidx < eff_end)
                    wbuf[sl] = jnp.where(valid, ex, 0.0)
                    ln = dstv[sl] - d0
                    lnbuf[sl] = jnp.where(valid, ln, jnp.int32(SUBN))

                def acc(e, _a):
                    w_e = wbuf[pl.ds(e, 16)][0]
                    ln_e = lnbuf[pl.ds(e, 16)][0]
                    dv = den[pl.ds(ln_e, 16)]
                    den[pl.ds(ln_e, 16)] = jnp.where(lane == 0, dv + w_e, dv)
                    for c in range(H // 16):
                        sl = pl.ds(c * 16, 16)
                        obuf[ln_e, sl] = obuf[ln_e, sl] + w_e * rows[e, sl]
                    return 0

                lax.fori_loop(0, C, acc, 0, unroll=4)

            @pl.when(nch > 0)
            def _pipeline():
                issue_idx(0, 0)
                wait_idx(0)
                issue_gath(0)
                issue_idx(1, jnp.minimum(1, nchm1))

                def outer(kk, _c):
                    for b in range(2):
                        j = 2 * kk + b
                        q = 1 - b
                        wait_idx(q)
                        issue_gath(q)
                        wait_gath(b)
                        issue_idx(b, jnp.minimum(j + 2, nchm1))
                        compute(b, j)
                    return 0

                lax.fori_loop(0, lax.div(nch + 1, 2), outer, 0)
                wait_gath(0)
                wait_idx(1)

            def recip(gq, _c):
                sl = pl.ds(gq * 16, 16)
                den[sl] = 1.0 / (den[sl] + 1e-16)
                return 0
            lax.fori_loop(0, (SUBN + 16) // 16, recip, 0)

            def scale(i, _c):
                inv = den[pl.ds(i, 16)][0]
                for c in range(H // 16):
                    sl = pl.ds(c * 16, 16)
                    obuf[i, sl] = obuf[i, sl] * inv
                return 0
            lax.fori_loop(0, SUBN, scale, 0)

            pltpu.sync_copy(obuf.at[pl.ds(0, SUBN)], out.at[pl.ds(d0, SUBN)])
            return 0

        lax.fori_loop(0, NSUB, subchunk, 0)

    return k


_att_msg_gat = _make_att_msg(False)
_att_msg_g = _make_att_msg(True)


def _make_seg_pool(att_mode):
    mesh = _mesh()
    GPW = G // NW

    @functools.partial(
        pl.kernel, mesh=mesh,
        out_type=jax.ShapeDtypeStruct((G, H), jnp.float32),
        scratch_types=[
            pltpu.VMEM((GPW + 17,), jnp.int32),
            pltpu.VMEM((32,), jnp.float32),
            pltpu.VMEM((C,), jnp.float32),
            pltpu.VMEM((C + 16,), jnp.float32),
            pltpu.VMEM((C, H), jnp.float32),
            pltpu.VMEM((32,), jnp.float32),
            pltpu.VMEM((GPW, H), jnp.float32),
            pltpu.SemaphoreType.DMA,
        ],
    )
    def k(xs, asn, adg, offh, out, offv, adgv, asv, wbuf, rows, den, obuf,
          sem):
        wid = lax.axis_index("s") * 2 + lax.axis_index("c")
        g0 = pl.multiple_of(wid * GPW, 8)
        pltpu.sync_copy(offh.at[pl.ds(g0, GPW + 1)], offv.at[pl.ds(0, GPW + 1)])
        if att_mode:
            pltpu.sync_copy(adg.at[pl.ds(g0, GPW)], adgv.at[pl.ds(0, GPW)])
        lane = lax.iota(jnp.int32, 16)

        def zb(i, _c):
            for c in range(H // 16):
                obuf[i, pl.ds(c * 16, 16)] = jnp.zeros((16,), jnp.float32)
            return 0
        lax.fori_loop(0, GPW, zb, 0)

        def graph(gi, _c):
            b0 = offv[pl.ds(gi, 16)][0]
            b1 = offv[pl.ds(gi + 1, 16)][0]
            if att_mode:
                ad_g = adgv[pl.ds(gi, 16)][0]
            else:
                ad_g = 0.0
            c0 = b0 - lax.rem(b0, 8)
            nch = lax.div(b1 - c0 + (C - 1), C)

            def chunk(j, dsum):
                e0 = pl.multiple_of(c0 + j * C, 8)
                pltpu.sync_copy(xs.at[pl.ds(e0, C)], rows)
                if att_mode:
                    pltpu.sync_copy(asn.at[pl.ds(e0, C)], asv)
                for g in range(C // 16):
                    sl = pl.ds(g * 16, 16)
                    nidx = e0 + g * 16 + lane
                    valid = (nidx >= b0) & (nidx < b1)
                    if att_mode:
                        al = asv[sl] + ad_g
                        al = jnp.where(al >= 0, al, NEG * al)
                        ex = jnp.exp(al)
                    else:
                        ex = jnp.ones((16,), jnp.float32)
                    wbuf[sl] = jnp.where(valid, ex, 0.0)

                def acc(e, da):
                    w_e = wbuf[pl.ds(e, 16)][0]
                    for c in range(H // 16):
                        sl = pl.ds(c * 16, 16)
                        obuf[gi, sl] = obuf[gi, sl] + w_e * rows[e, sl]
                    return da + w_e

                return lax.fori_loop(0, C, acc, dsum, unroll=4)

            dsum = lax.fori_loop(0, nch, chunk, 0.0)
            dv = den[pl.ds(gi, 16)]
            den[pl.ds(gi, 16)] = jnp.where(lane == 0, dsum, dv)
            return 0

        lax.fori_loop(0, GPW, graph, 0)

        if att_mode:
            dv = den[pl.ds(0, 16)]
            den[pl.ds(0, 16)] = 1.0 / (dv + 1e-16)

            def scale(i, _c):
                inv = den[pl.ds(i, 16)][0]
                for c in range(H // 16):
                    sl = pl.ds(c * 16, 16)
                    obuf[i, sl] = obuf[i, sl] * inv
                return 0
            lax.fori_loop(0, GPW, scale, 0)

        pltpu.sync_copy(obuf, out.at[pl.ds(g0, GPW)])

    return k


_seg_pool_sum = _make_seg_pool(False)
_seg_pool_att = _make_seg_pool(True)


def kernel(x, edge_index, edge_attr, batch, params):
    p = params
    src = edge_index[0]
    dst = edge_index[1]

    eiota = lax.iota(jnp.int32, N_EDGES)
    dst_s, src_s, perm = lax.sort([dst, src, eiota], num_keys=1)
    counts = jax.ops.segment_sum(jnp.ones((N_EDGES,), jnp.int32), dst,
                                 num_segments=N_NODES)
    offsets = jnp.concatenate([jnp.zeros((1,), jnp.int32),
                               jnp.cumsum(counts, dtype=jnp.int32)])
    offsets = jnp.pad(offsets, (0, OFFPAD - (N_NODES + 1)),
                      constant_values=N_EDGES)
    bcounts = jax.ops.segment_sum(jnp.ones((N_NODES,), jnp.int32), batch,
                                  num_segments=G)
    boffsets = jnp.concatenate([jnp.zeros((1,), jnp.int32),
                                jnp.cumsum(bcounts, dtype=jnp.int32)])
    boffsets = jnp.pad(boffsets, (0, 32), constant_values=N_NODES)

    epad = EPAD - N_EDGES
    src_s = jnp.pad(src_s, (0, epad))
    dst_s = jnp.pad(dst_s, (0, epad), constant_values=N_NODES - 8)
    perm = jnp.pad(perm, (0, epad))
    src_o = jnp.pad(src, (0, epad))
    ea = jnp.pad(edge_attr, ((0, epad), (0, 0)))

    xpad = jnp.pad(x, ((0, NPAD - N_NODES), (0, 1)))

    lin1_wt = jnp.pad(p['lin1_W'].T, ((0, 1), (0, 0)))
    w1x_t = p['g_lin1_W'][:, :H].T
    w1e_t = p['g_lin1_W'][:, H:].T
    g2_t = p['g_lin2_W'].T
    vad = p['mol_lin_W'].T @ p['mol_att_dst']

    def r2(v):
        return v.reshape(1, -1)

    def c2(v):
        return v.reshape(-1, 1)

    x0 = _tc_lin1(xpad, lin1_wt, r2(p['lin1_b']))
    xw1, xg2, ar1 = _tc_pre_g(x0, w1x_t, g2_t, c2(p['g_att_r']))

    xw1g = _sc_gather_rows(xw1, src_o)
    b_edge = _tc_edge_alpha(xw1g, ea, w1e_t, c2(p['g_att_l']))

    msg = _att_msg_g(xg2, b_edge.reshape(-1), ar1.reshape(-1),
                     src_s, dst_s, perm, offsets)
    xcur = _tc_gru(msg, x0, r2(p['g_bias']),
                   p['gru0_Wih'].T, p['gru0_Whh'].T,
                   r2(p['gru0_bih']), r2(p['gru0_bhh']))

    for l in range(2):
        xt, a_s, a_d = _tc_gat_pre(xcur, p['a%d_lin_W' % l].T,
                                   c2(p['a%d_att_src' % l]),
                                   c2(p['a%d_att_dst' % l]))
        msg = _att_msg_gat(xt, a_s.reshape(-1), a_d.reshape(-1),
                           src_s, dst_s, perm, offsets)
        gname = 'gru%d' % (l + 1)
        xcur = _tc_gru(msg, xcur, r2(p['a%d_bias' % l]),
                       p[gname + '_Wih'].T, p[gname + '_Whh'].T,
                       r2(p[gname + '_bih']), r2(p[gname + '_bhh']))

    zeros_n = jnp.zeros((NPAD,), jnp.float32)
    pooled = _seg_pool_sum(xcur, zeros_n, jnp.zeros((G,), jnp.float32),
                           boffsets)
    out, ad = _tc_pool_init(pooled, c2(vad))
    xs, asn = _tc_mol_pre(xcur, p['mol_lin_W'].T, c2(p['mol_att_src']))
    for _t in range(2):
        h = _seg_pool_att(xs, asn.reshape(-1), ad.reshape(-1), boffsets)
        out, ad = _tc_gru_mol(h, out, r2(p['mol_bias']),
                              p['gru3_Wih'].T, p['gru3_Whh'].T,
                              r2(p['gru3_bih']), r2(p['gru3_bhh']),
                              c2(vad))

    return _tc_head(out, p['lin2_W'].T, r2(p['lin2_b']),
                    p['head_W1'].T, r2(p['head_b1']),
                    p['head_W2'].T, r2(p['head_b2']))
